# baseline (speedup 1.0000x reference)
import jax
import jax.numpy as jnp
from jax import lax
from jax.experimental import pallas as pl
from jax.experimental.pallas import tpu as pltpu
from jax.experimental.pallas import tpu_sc as plsc
import functools

VOCAB = 1000
HIDDEN = 128
BATCH = 1024
SEQ = 50

NW = 32
BATCH_W = BATCH // NW


_sc_mesh = plsc.VectorSubcoreMesh(core_axis_name="c", subcore_axis_name="s")


@functools.partial(
    pl.kernel,
    mesh=_sc_mesh,
    out_type=jax.ShapeDtypeStruct((BATCH, SEQ, HIDDEN), jnp.float32),
    scratch_types=[
        pltpu.VMEM((SEQ,), jnp.int32),
        pltpu.VMEM((SEQ,), jnp.int32),
        pltpu.VMEM((2, SEQ, HIDDEN), jnp.float32),
        pltpu.SemaphoreType.DMA,
        pltpu.SemaphoreType.DMA,
        pltpu.SemaphoreType.DMA,
        pltpu.SemaphoreType.DMA,
        pltpu.SemaphoreType.DMA,
        pltpu.SemaphoreType.DMA,
    ],
)
def _sc_gather(emb, ids, out, idxA, idxB, rows_v, g0, g1, s0, s1, i0, i1):
    cid = lax.axis_index("c")
    sid = lax.axis_index("s")
    wid = sid * 2 + cid
    base = wid * BATCH_W

    def idx_fetch(c, idx, sem):
        pltpu.async_copy(ids.at[base + c], idx, sem)

    def idx_wait(c, idx, sem):
        pltpu.make_async_copy(ids.at[base + c], idx, sem).wait()

    def gather(idx, slot, sem):
        pltpu.async_copy(emb.at[idx], rows_v.at[slot], sem)

    def gather_wait(idx, slot, sem):
        pltpu.make_async_copy(emb.at[idx], rows_v.at[slot], sem).wait()

    def scatter(c, slot, sem):
        pltpu.async_copy(rows_v.at[slot], out.at[base + c], sem)

    def scatter_wait(c, slot, sem):
        pltpu.make_async_copy(rows_v.at[slot], out.at[base + c], sem).wait()

    pltpu.sync_copy(ids.at[base], idxA)
    pltpu.sync_copy(ids.at[base + 1], idxB)
    gather(idxA, 0, g0)
    gather(idxB, 1, g1)

    def body(g, carry):
        c0 = 2 * g
        c1 = c0 + 1
        gather_wait(idxA, 0, g0)
        idx_fetch(c0 + 2, idxA, i0)
        scatter(c0, 0, s0)
        gather_wait(idxB, 1, g1)
        idx_fetch(c1 + 2, idxB, i1)
        scatter(c1, 1, s1)
        scatter_wait(c0, 0, s0)
        idx_wait(c0 + 2, idxA, i0)
        gather(idxA, 0, g0)
        scatter_wait(c1, 1, s1)
        idx_wait(c1 + 2, idxB, i1)
        gather(idxB, 1, g1)
        return carry

    lax.fori_loop(0, BATCH_W // 2 - 1, body, 0, unroll=False)

    gather_wait(idxA, 0, g0)
    scatter(BATCH_W - 2, 0, s0)
    gather_wait(idxB, 1, g1)
    scatter(BATCH_W - 1, 1, s1)
    scatter_wait(BATCH_W - 2, 0, s0)
    scatter_wait(BATCH_W - 1, 1, s1)


def _mm_body(wt_ref, b_ref, emb_ref, out_ref):
    s = pl.program_id(0)
    e = emb_ref[:, s, :]
    out_ref[0] = (
        jax.lax.dot_general(
            wt_ref[...], e.astype(jnp.bfloat16), (((1,), (1,)), ((), ())),
            preferred_element_type=jnp.float32,
        )
        + b_ref[...]
    )


def _matmul(WT, b_col, emb_g):
    return pl.pallas_call(
        _mm_body,
        grid=(SEQ,),
        compiler_params=pltpu.CompilerParams(
            vmem_limit_bytes=48 * 1024 * 1024
        ),
        in_specs=[
            pl.BlockSpec((VOCAB, HIDDEN), lambda s: (0, 0)),
            pl.BlockSpec((VOCAB, 1), lambda s: (0, 0)),
            pl.BlockSpec((BATCH, SEQ, HIDDEN), lambda s: (0, 0, 0)),
        ],
        out_specs=pl.BlockSpec((1, VOCAB, BATCH), lambda s: (s, 0, 0)),
        out_shape=jax.ShapeDtypeStruct((SEQ, VOCAB, BATCH), jnp.float32),
    )(WT.astype(jnp.bfloat16), b_col, emb_g)


def kernel(input_ids, embedding, W, b):
    ids = input_ids.astype(jnp.int32)
    emb_g = _sc_gather(embedding, ids)
    out_t = _matmul(W.T, b.reshape(VOCAB, 1), emb_g)
    return out_t.transpose(2, 0, 1)


# speedup vs baseline: 1.0367x; 1.0367x over previous
import jax
import jax.numpy as jnp
from jax import lax
from jax.experimental import pallas as pl
from jax.experimental.pallas import tpu as pltpu
from jax.experimental.pallas import tpu_sc as plsc
import functools

VOCAB = 1000
HIDDEN = 128
BATCH = 1024
SEQ = 50

NW = 32
BATCH_W = BATCH // NW
GB = 4
NCHUNK = BATCH_W // GB


_sc_mesh = plsc.VectorSubcoreMesh(core_axis_name="c", subcore_axis_name="s")


@functools.partial(
    pl.kernel,
    mesh=_sc_mesh,
    out_type=jax.ShapeDtypeStruct((BATCH, SEQ, HIDDEN), jnp.float32),
    scratch_types=[
        pltpu.VMEM((GB, SEQ), jnp.int32),
        pltpu.VMEM((GB, SEQ), jnp.int32),
        pltpu.VMEM((2, GB, SEQ, HIDDEN), jnp.float32),
        pltpu.SemaphoreType.DMA,
        pltpu.SemaphoreType.DMA,
        pltpu.SemaphoreType.DMA,
        pltpu.SemaphoreType.DMA,
        pltpu.SemaphoreType.DMA,
        pltpu.SemaphoreType.DMA,
    ],
)
def _sc_gather(emb, ids, out, idxA, idxB, rows_v, g0, g1, s0, s1, i0, i1):
    cid = lax.axis_index("c")
    sid = lax.axis_index("s")
    wid = sid * 2 + cid
    base = wid * BATCH_W

    def idx_fetch(c, idx, sem):
        pltpu.async_copy(ids.at[pl.ds(base + c * GB, GB)], idx, sem)

    def idx_wait(c, idx, sem):
        pltpu.make_async_copy(ids.at[pl.ds(base + c * GB, GB)], idx, sem).wait()

    def gather(idx, slot, sem):
        for k in range(GB):
            pltpu.async_copy(emb.at[idx.at[k]], rows_v.at[slot, k], sem)

    def gather_wait(idx, slot, sem):
        for k in range(GB):
            pltpu.make_async_copy(
                emb.at[idx.at[k]], rows_v.at[slot, k], sem
            ).wait()

    def scatter(c, slot, sem):
        pltpu.async_copy(
            rows_v.at[slot], out.at[pl.ds(base + c * GB, GB)], sem
        )

    def scatter_wait(c, slot, sem):
        pltpu.make_async_copy(
            rows_v.at[slot], out.at[pl.ds(base + c * GB, GB)], sem
        ).wait()

    pltpu.sync_copy(ids.at[pl.ds(base, GB)], idxA)
    pltpu.sync_copy(ids.at[pl.ds(base + GB, GB)], idxB)
    gather(idxA, 0, g0)
    gather(idxB, 1, g1)

    def body(g, carry):
        c0 = 2 * g
        c1 = c0 + 1
        gather_wait(idxA, 0, g0)
        idx_fetch(c0 + 2, idxA, i0)
        scatter(c0, 0, s0)
        gather_wait(idxB, 1, g1)
        idx_fetch(c1 + 2, idxB, i1)
        scatter(c1, 1, s1)
        scatter_wait(c0, 0, s0)
        idx_wait(c0 + 2, idxA, i0)
        gather(idxA, 0, g0)
        scatter_wait(c1, 1, s1)
        idx_wait(c1 + 2, idxB, i1)
        gather(idxB, 1, g1)
        return carry

    lax.fori_loop(0, NCHUNK // 2 - 1, body, 0, unroll=False)

    gather_wait(idxA, 0, g0)
    scatter(NCHUNK - 2, 0, s0)
    gather_wait(idxB, 1, g1)
    scatter(NCHUNK - 1, 1, s1)
    scatter_wait(NCHUNK - 2, 0, s0)
    scatter_wait(NCHUNK - 1, 1, s1)


def _mm_body(wt_ref, b_ref, emb_ref, out_ref):
    s = pl.program_id(0)
    e = emb_ref[:, s, :]
    out_ref[0] = (
        jax.lax.dot_general(
            wt_ref[...], e, (((1,), (1,)), ((), ())),
            preferred_element_type=jnp.float32,
        )
        + b_ref[...]
    )


def _matmul(WT, b_col, emb_g):
    return pl.pallas_call(
        _mm_body,
        grid=(SEQ,),
        compiler_params=pltpu.CompilerParams(
            vmem_limit_bytes=48 * 1024 * 1024
        ),
        in_specs=[
            pl.BlockSpec((VOCAB, HIDDEN), lambda s: (0, 0)),
            pl.BlockSpec((VOCAB, 1), lambda s: (0, 0)),
            pl.BlockSpec((BATCH, SEQ, HIDDEN), lambda s: (0, 0, 0)),
        ],
        out_specs=pl.BlockSpec((1, VOCAB, BATCH), lambda s: (s, 0, 0)),
        out_shape=jax.ShapeDtypeStruct((SEQ, VOCAB, BATCH), jnp.float32),
    )(WT, b_col, emb_g)


def kernel(input_ids, embedding, W, b):
    ids = input_ids.astype(jnp.int32)
    emb_g = _sc_gather(embedding, ids)
    out_t = _matmul(W.T, b.reshape(VOCAB, 1), emb_g)
    return out_t.transpose(2, 0, 1)


# speedup vs baseline: 1.1072x; 1.0680x over previous
import jax
import jax.numpy as jnp
from jax import lax
from jax.experimental import pallas as pl
from jax.experimental.pallas import tpu as pltpu
from jax.experimental.pallas import tpu_sc as plsc
import functools

VOCAB = 1000
HIDDEN = 128
BATCH = 1024
SEQ = 50

NW = 32
BATCH_W = BATCH // NW
GB = 4
NCHUNK = BATCH_W // GB


_sc_mesh = plsc.VectorSubcoreMesh(core_axis_name="c", subcore_axis_name="s")


@functools.partial(
    pl.kernel,
    mesh=_sc_mesh,
    out_type=jax.ShapeDtypeStruct((BATCH, SEQ, HIDDEN), jnp.float32),
    scratch_types=[
        pltpu.VMEM((GB, SEQ), jnp.int32),
        pltpu.VMEM((GB, SEQ), jnp.int32),
        pltpu.VMEM((2, GB, SEQ, HIDDEN), jnp.float32),
        pltpu.SemaphoreType.DMA,
        pltpu.SemaphoreType.DMA,
        pltpu.SemaphoreType.DMA,
        pltpu.SemaphoreType.DMA,
        pltpu.SemaphoreType.DMA,
        pltpu.SemaphoreType.DMA,
    ],
)
def _sc_gather(emb, ids, out, idxA, idxB, rows_v, g0, g1, s0, s1, i0, i1):
    cid = lax.axis_index("c")
    sid = lax.axis_index("s")
    wid = sid * 2 + cid
    base = wid * BATCH_W

    def idx_fetch(c, idx, sem):
        pltpu.async_copy(ids.at[pl.ds(base + c * GB, GB)], idx, sem)

    def idx_wait(c, idx, sem):
        pltpu.make_async_copy(ids.at[pl.ds(base + c * GB, GB)], idx, sem).wait()

    def gather(idx, slot, sem):
        for k in range(GB):
            pltpu.async_copy(emb.at[idx.at[k]], rows_v.at[slot, k], sem)

    def gather_wait(idx, slot, sem):
        for k in range(GB):
            pltpu.make_async_copy(
                emb.at[idx.at[k]], rows_v.at[slot, k], sem
            ).wait()

    def scatter(c, slot, sem):
        pltpu.async_copy(
            rows_v.at[slot], out.at[pl.ds(base + c * GB, GB)], sem
        )

    def scatter_wait(c, slot, sem):
        pltpu.make_async_copy(
            rows_v.at[slot], out.at[pl.ds(base + c * GB, GB)], sem
        ).wait()

    pltpu.sync_copy(ids.at[pl.ds(base, GB)], idxA)
    pltpu.sync_copy(ids.at[pl.ds(base + GB, GB)], idxB)
    gather(idxA, 0, g0)
    gather(idxB, 1, g1)

    def body(g, carry):
        c0 = 2 * g
        c1 = c0 + 1
        gather_wait(idxA, 0, g0)
        idx_fetch(c0 + 2, idxA, i0)
        scatter(c0, 0, s0)
        gather_wait(idxB, 1, g1)
        idx_fetch(c1 + 2, idxB, i1)
        scatter(c1, 1, s1)
        scatter_wait(c0, 0, s0)
        idx_wait(c0 + 2, idxA, i0)
        gather(idxA, 0, g0)
        scatter_wait(c1, 1, s1)
        idx_wait(c1 + 2, idxB, i1)
        gather(idxB, 1, g1)
        return carry

    lax.fori_loop(0, NCHUNK // 2 - 1, body, 0, unroll=False)

    gather_wait(idxA, 0, g0)
    scatter(NCHUNK - 2, 0, s0)
    gather_wait(idxB, 1, g1)
    scatter(NCHUNK - 1, 1, s1)
    scatter_wait(NCHUNK - 2, 0, s0)
    scatter_wait(NCHUNK - 1, 1, s1)


SB = 2


def _mm_body(wt_ref, b_ref, emb_ref, out_ref):
    g = pl.program_id(0)
    for j in range(SB):
        e = emb_ref[:, g * SB + j, :]
        out_ref[j] = (
            jax.lax.dot_general(
                wt_ref[...], e, (((1,), (1,)), ((), ())),
                preferred_element_type=jnp.float32,
            )
            + b_ref[...]
        )


def _matmul(WT, b_col, emb_g):
    return pl.pallas_call(
        _mm_body,
        grid=(SEQ // SB,),
        compiler_params=pltpu.CompilerParams(
            vmem_limit_bytes=48 * 1024 * 1024
        ),
        in_specs=[
            pl.BlockSpec((VOCAB, HIDDEN), lambda s: (0, 0)),
            pl.BlockSpec((VOCAB, 1), lambda s: (0, 0)),
            pl.BlockSpec((BATCH, SEQ, HIDDEN), lambda s: (0, 0, 0)),
        ],
        out_specs=pl.BlockSpec((SB, VOCAB, BATCH), lambda s: (s, 0, 0)),
        out_shape=jax.ShapeDtypeStruct((SEQ, VOCAB, BATCH), jnp.float32),
    )(WT, b_col, emb_g)


def kernel(input_ids, embedding, W, b):
    ids = input_ids.astype(jnp.int32)
    emb_g = _sc_gather(embedding, ids)
    out_t = _matmul(W.T, b.reshape(VOCAB, 1), emb_g)
    return out_t.transpose(2, 0, 1)
